# Initial kernel scaffold; baseline (speedup 1.0000x reference)
import jax
import jax.numpy as jnp
from jax.experimental import pallas as pl


def kernel(anchor_input_ids, anchor_attention_mask, positive_input_ids, positive_attention_mask, negative_input_ids, negative_attention_mask, emb_table, fc_W, fc_b):
    raise NotImplementedError("write your pallas kernel here")


# speedup vs baseline: 8.1902x; 8.1902x over previous
import functools

import jax
import jax.numpy as jnp
from jax import lax
from jax.experimental import pallas as pl
from jax.experimental.pallas import tpu as pltpu
from jax.experimental.pallas import tpu_sc as plsc

VOCAB = 30522
EMBED = 128
HIDDEN = 64
B = 1024
S = 512
ROWS = 3 * B
CHUNK = 128
NCHUNK = S // CHUNK

BLK = 2048
VOCAB_PAD = ((VOCAB + BLK - 1) // BLK) * BLK


def _proj_body(tab_ref, w_ref, out_ref):
    out_ref[...] = jnp.dot(
        tab_ref[...], w_ref[...],
        preferred_element_type=jnp.float32,
        precision=lax.Precision.HIGHEST,
    )


def _project_table(table_padded, fc_W):
    return pl.pallas_call(
        _proj_body,
        grid=(VOCAB_PAD // BLK,),
        in_specs=[
            pl.BlockSpec((BLK, EMBED), lambda i: (i, 0)),
            pl.BlockSpec((EMBED, HIDDEN), lambda i: (0, 0)),
        ],
        out_specs=pl.BlockSpec((BLK, HIDDEN), lambda i: (i, 0)),
        out_shape=jax.ShapeDtypeStruct((VOCAB_PAD, HIDDEN), jnp.float32),
    )(table_padded, fc_W)


def _make_sc_pool():
    info = plsc.get_sparse_core_info()
    nc, ns = info.num_cores, info.num_subcores
    nw = nc * ns
    rpw = ROWS // nw

    mesh = plsc.VectorSubcoreMesh(core_axis_name="c", subcore_axis_name="s")

    @functools.partial(
        pl.kernel,
        mesh=mesh,
        out_type=jax.ShapeDtypeStruct((ROWS, HIDDEN), jnp.float32),
        scratch_types=[
            pltpu.VMEM((NCHUNK, CHUNK), jnp.int32),
            pltpu.VMEM((CHUNK, HIDDEN), jnp.float32),
            pltpu.VMEM((rpw, HIDDEN), jnp.float32),
            pltpu.VMEM((HIDDEN,), jnp.float32),
            pltpu.SemaphoreType.DMA,
        ],
        compiler_params=pltpu.CompilerParams(use_tc_tiling_on_sc=False),
    )
    def sc_pool(ids_hbm, proj_hbm, bias_hbm, out_hbm,
                idx_v, gat_v, out_v, bias_v, sem):
        wid = lax.axis_index("s") * nc + lax.axis_index("c")
        base = wid * rpw
        pltpu.sync_copy(bias_hbm, bias_v)

        def row_body(r, _):
            row = base + r
            pltpu.sync_copy(ids_hbm.at[row], idx_v)

            accs = tuple(jnp.zeros((16,), jnp.float32) for _ in range(4))
            for c in range(NCHUNK):
                pltpu.async_copy(proj_hbm.at[idx_v.at[c]], gat_v, sem).wait()

                def acc_body(i, carry):
                    return tuple(
                        carry[q] + gat_v[i, pl.ds(q * 16, 16)]
                        for q in range(4)
                    )
                accs = lax.fori_loop(0, CHUNK, acc_body, accs)

            inv = jnp.float32(1.0 / S)
            for q in range(4):
                val = jnp.maximum(
                    accs[q] * inv + bias_v[pl.ds(q * 16, 16)], 0.0)
                out_v[r, pl.ds(q * 16, 16)] = val
            return 0

        lax.fori_loop(0, rpw, row_body, 0)
        pltpu.sync_copy(out_v, out_hbm.at[pl.ds(base, rpw)])

    return sc_pool


def kernel(anchor_input_ids, anchor_attention_mask,
           positive_input_ids, positive_attention_mask,
           negative_input_ids, negative_attention_mask,
           emb_table, fc_W, fc_b):
    ids = jnp.concatenate(
        [anchor_input_ids, positive_input_ids, negative_input_ids], axis=0
    ).astype(jnp.int32).reshape(ROWS, NCHUNK, CHUNK)
    table_padded = jnp.pad(emb_table, ((0, VOCAB_PAD - VOCAB), (0, 0)))
    proj = _project_table(table_padded, fc_W)
    return _make_sc_pool()(ids, proj, fc_b)


# speedup vs baseline: 12.0206x; 1.4677x over previous
import functools

import jax
import jax.numpy as jnp
from jax import lax
from jax.experimental import pallas as pl
from jax.experimental.pallas import tpu as pltpu
from jax.experimental.pallas import tpu_sc as plsc

VOCAB = 30522
EMBED = 128
HIDDEN = 64
B = 1024
S = 512
ROWS = 3 * B
CHUNK = 128
NCHUNK = S // CHUNK

BLK = 2048
VOCAB_PAD = ((VOCAB + BLK - 1) // BLK) * BLK


def _proj_body(tab_ref, w_ref, out_ref):
    out_ref[...] = jnp.dot(
        tab_ref[...], w_ref[...],
        preferred_element_type=jnp.float32,
        precision=lax.Precision.HIGHEST,
    )


def _project_table(table_padded, fc_W):
    return pl.pallas_call(
        _proj_body,
        grid=(VOCAB_PAD // BLK,),
        in_specs=[
            pl.BlockSpec((BLK, EMBED), lambda i: (i, 0)),
            pl.BlockSpec((EMBED, HIDDEN), lambda i: (0, 0)),
        ],
        out_specs=pl.BlockSpec((BLK, HIDDEN), lambda i: (i, 0)),
        out_shape=jax.ShapeDtypeStruct((VOCAB_PAD, HIDDEN), jnp.float32),
    )(table_padded, fc_W)


def _make_sc_pool():
    info = plsc.get_sparse_core_info()
    nc, ns = info.num_cores, info.num_subcores
    nw = nc * ns
    rpw = ROWS // nw

    mesh = plsc.VectorSubcoreMesh(core_axis_name="c", subcore_axis_name="s")

    @functools.partial(
        pl.kernel,
        mesh=mesh,
        out_type=jax.ShapeDtypeStruct((ROWS, HIDDEN), jnp.float32),
        scratch_types=[
            pltpu.VMEM((rpw, NCHUNK, CHUNK), jnp.int32),
            pltpu.VMEM((CHUNK, HIDDEN), jnp.float32),
            pltpu.VMEM((CHUNK, HIDDEN), jnp.float32),
            pltpu.VMEM((rpw, HIDDEN), jnp.float32),
            pltpu.VMEM((HIDDEN,), jnp.float32),
            pltpu.SemaphoreType.DMA,
            pltpu.SemaphoreType.DMA,
        ],
        compiler_params=pltpu.CompilerParams(use_tc_tiling_on_sc=False),
    )
    def sc_pool(ids_hbm, proj_hbm, bias_hbm, out_hbm,
                idx_v, g0, g1, out_v, bias_v, sem0, sem1):
        wid = lax.axis_index("s") * nc + lax.axis_index("c")
        base = wid * rpw
        pltpu.sync_copy(bias_hbm, bias_v)
        pltpu.sync_copy(ids_hbm.at[pl.ds(base, rpw)], idx_v)
        pltpu.async_copy(proj_hbm.at[idx_v.at[0, 0]], g0, sem0)
        dummy = proj_hbm.at[pl.ds(0, CHUNK)]

        def row_body(r, _):
            accs = tuple(jnp.zeros((16,), jnp.float32) for _ in range(4))
            for c in range(NCHUNK):
                gcur, scur = (g0, sem0) if c % 2 == 0 else (g1, sem1)
                gnxt, snxt = (g1, sem1) if c % 2 == 0 else (g0, sem0)
                pltpu.make_async_copy(dummy, gcur, scur).wait()
                if c + 1 < NCHUNK:
                    pltpu.async_copy(
                        proj_hbm.at[idx_v.at[r, c + 1]], gnxt, snxt)
                else:
                    @pl.when(r + 1 < rpw)
                    def _():
                        pltpu.async_copy(
                            proj_hbm.at[idx_v.at[r + 1, 0]], gnxt, snxt)

                @plsc.parallel_loop(0, CHUNK, unroll=8, carry=accs)
                def accs(i, carry):
                    return tuple(
                        carry[q] + gcur[i, pl.ds(q * 16, 16)]
                        for q in range(4)
                    )

            inv = jnp.float32(1.0 / S)
            for q in range(4):
                val = jnp.maximum(
                    accs[q] * inv + bias_v[pl.ds(q * 16, 16)], 0.0)
                out_v[r, pl.ds(q * 16, 16)] = val
            return 0

        lax.fori_loop(0, rpw, row_body, 0)
        pltpu.sync_copy(out_v, out_hbm.at[pl.ds(base, rpw)])

    return sc_pool


def kernel(anchor_input_ids, anchor_attention_mask,
           positive_input_ids, positive_attention_mask,
           negative_input_ids, negative_attention_mask,
           emb_table, fc_W, fc_b):
    ids = jnp.concatenate(
        [anchor_input_ids, positive_input_ids, negative_input_ids], axis=0
    ).astype(jnp.int32).reshape(ROWS, NCHUNK, CHUNK)
    table_padded = jnp.pad(emb_table, ((0, VOCAB_PAD - VOCAB), (0, 0)))
    proj = _project_table(table_padded, fc_W)
    return _make_sc_pool()(ids, proj, fc_b)


# speedup vs baseline: 28.7693x; 2.3933x over previous
import functools

import jax
import jax.numpy as jnp
from jax import lax
from jax.experimental import pallas as pl
from jax.experimental.pallas import tpu as pltpu
from jax.experimental.pallas import tpu_sc as plsc

VOCAB = 30522
EMBED = 128
HIDDEN = 64
PACKED = HIDDEN // 2
B = 1024
S = 512
ROWS = 3 * B
CHUNK = 128
NCHUNK = S // CHUNK
NBUF = 4

BLK = 2048


def _round_to_bf16_bits(x):
    u = lax.bitcast_convert_type(x, jnp.int32)
    r = u + jnp.int32(0x7FFF) + ((u >> 16) & jnp.int32(1))
    return r & jnp.int32(-65536)


def _proj_body(tab_ref, we_ref, wo_ref, out_ref):
    pe = jnp.dot(tab_ref[...], we_ref[...],
                 preferred_element_type=jnp.float32,
                 precision=lax.Precision.HIGHEST)
    po = jnp.dot(tab_ref[...], wo_ref[...],
                 preferred_element_type=jnp.float32,
                 precision=lax.Precision.HIGHEST)
    lo = lax.shift_right_logical(_round_to_bf16_bits(pe), 16)
    hi = _round_to_bf16_bits(po)
    out_ref[...] = hi | lo


def _project_table(table, we, wo):
    grid = (VOCAB + BLK - 1) // BLK
    return pl.pallas_call(
        _proj_body,
        grid=(grid,),
        in_specs=[
            pl.BlockSpec((BLK, EMBED), lambda i: (i, 0)),
            pl.BlockSpec((EMBED, PACKED), lambda i: (0, 0)),
            pl.BlockSpec((EMBED, PACKED), lambda i: (0, 0)),
        ],
        out_specs=pl.BlockSpec((BLK, PACKED), lambda i: (i, 0)),
        out_shape=jax.ShapeDtypeStruct((VOCAB, PACKED), jnp.int32),
    )(table, we, wo)


def _make_sc_pool():
    info = plsc.get_sparse_core_info()
    nc, ns = info.num_cores, info.num_subcores
    nw = nc * ns
    rpb = B // nw
    rpw = 3 * rpb

    mesh = plsc.VectorSubcoreMesh(core_axis_name="c", subcore_axis_name="s")

    @functools.partial(
        pl.kernel,
        mesh=mesh,
        out_type=jax.ShapeDtypeStruct((ROWS, HIDDEN), jnp.float32),
        scratch_types=[
            pltpu.VMEM((rpw, NCHUNK, CHUNK), jnp.int32),
            *[pltpu.VMEM((CHUNK, PACKED), jnp.int32) for _ in range(NBUF)],
            pltpu.VMEM((rpw, HIDDEN), jnp.float32),
            pltpu.VMEM((HIDDEN,), jnp.float32),
            *[pltpu.SemaphoreType.DMA for _ in range(NBUF)],
        ],
        compiler_params=pltpu.CompilerParams(
            use_tc_tiling_on_sc=False, needs_layout_passes=False),
    )
    def sc_pool(a_hbm, p_hbm, n_hbm, proj_hbm, bias_hbm, out_hbm,
                idx_v, g0, g1, g2, g3, out_v, bias_v, s0, s1, s2, s3):
        wid = lax.axis_index("s") * nc + lax.axis_index("c")
        gbufs = (g0, g1, g2, g3)
        sems = (s0, s1, s2, s3)
        pltpu.sync_copy(bias_hbm, bias_v)
        for t, ids_hbm in enumerate((a_hbm, p_hbm, n_hbm)):
            pltpu.async_copy(ids_hbm.at[pl.ds(wid * rpb, rpb)],
                             idx_v.at[pl.ds(t * rpb, rpb)], sems[t])
        for t in range(3):
            pltpu.make_async_copy(
                a_hbm.at[pl.ds(0, rpb)],
                idx_v.at[pl.ds(t * rpb, rpb)], sems[t]).wait()
        for c in range(NBUF - 1):
            pltpu.async_copy(proj_hbm.at[idx_v.at[0, c]], gbufs[c], sems[c])
        dummy = proj_hbm.at[pl.ds(0, CHUNK)]

        def row_body(r, _):
            accs = tuple(jnp.zeros((16,), jnp.float32) for _ in range(4))
            for c in range(NCHUNK):
                gcur, scur = gbufs[c], sems[c]
                pltpu.make_async_copy(dummy, gcur, scur).wait()
                if c == 0:
                    pltpu.async_copy(
                        proj_hbm.at[idx_v.at[r, NCHUNK - 1]],
                        gbufs[NCHUNK - 1], sems[NCHUNK - 1])
                else:
                    @pl.when(r + 1 < rpw)
                    def _():
                        pltpu.async_copy(
                            proj_hbm.at[idx_v.at[r + 1, c - 1]],
                            gbufs[c - 1], sems[c - 1])

                @plsc.parallel_loop(0, CHUNK, unroll=16, carry=accs)
                def accs(i, carry):
                    a0, a1, a2, a3 = carry
                    w0 = gcur[i, pl.ds(0, 16)]
                    w1 = gcur[i, pl.ds(16, 16)]
                    a0 += plsc.bitcast(w0 << 16, jnp.float32)
                    a1 += plsc.bitcast(w0 & jnp.int32(-65536), jnp.float32)
                    a2 += plsc.bitcast(w1 << 16, jnp.float32)
                    a3 += plsc.bitcast(w1 & jnp.int32(-65536), jnp.float32)
                    return a0, a1, a2, a3

            inv = jnp.float32(1.0 / S)
            for q in range(4):
                val = jnp.maximum(
                    accs[q] * inv + bias_v[pl.ds(q * 16, 16)], 0.0)
                out_v[r, pl.ds(q * 16, 16)] = val
            return 0

        lax.fori_loop(0, rpw, row_body, 0)
        for t in range(3):
            pltpu.sync_copy(out_v.at[pl.ds(t * rpb, rpb)],
                            out_hbm.at[pl.ds(t * B + wid * rpb, rpb)])

    return sc_pool


def kernel(anchor_input_ids, anchor_attention_mask,
           positive_input_ids, positive_attention_mask,
           negative_input_ids, negative_attention_mask,
           emb_table, fc_W, fc_b):
    def prep(ids):
        return ids.astype(jnp.int32).reshape(B, NCHUNK, CHUNK)

    we = jnp.concatenate([fc_W[:, 0:16], fc_W[:, 32:48]], axis=1)
    wo = jnp.concatenate([fc_W[:, 16:32], fc_W[:, 48:64]], axis=1)
    proj = _project_table(emb_table, we, wo)
    return _make_sc_pool()(
        prep(anchor_input_ids), prep(positive_input_ids),
        prep(negative_input_ids), proj, fc_b)


# speedup vs baseline: 30.9062x; 1.0743x over previous
import functools

import jax
import jax.numpy as jnp
from jax import lax
from jax.experimental import pallas as pl
from jax.experimental.pallas import tpu as pltpu
from jax.experimental.pallas import tpu_sc as plsc

VOCAB = 30522
EMBED = 128
HIDDEN = 64
PACKED = HIDDEN // 2
B = 1024
S = 512
ROWS = 3 * B
CHUNK = 128
NCHUNK = S // CHUNK
NBUF = 4

BLK = 2048


def _round_to_bf16_bits(x):
    u = lax.bitcast_convert_type(x, jnp.int32)
    r = u + jnp.int32(0x7FFF) + ((u >> 16) & jnp.int32(1))
    return r & jnp.int32(-65536)


def _proj_body(tab_ref, we_ref, wo_ref, out_ref):
    pe = jnp.dot(tab_ref[...], we_ref[...],
                 preferred_element_type=jnp.float32)
    po = jnp.dot(tab_ref[...], wo_ref[...],
                 preferred_element_type=jnp.float32)
    lo = lax.shift_right_logical(_round_to_bf16_bits(pe), 16)
    hi = _round_to_bf16_bits(po)
    out_ref[...] = hi | lo


def _project_table(table, we, wo):
    grid = (VOCAB + BLK - 1) // BLK
    return pl.pallas_call(
        _proj_body,
        grid=(grid,),
        in_specs=[
            pl.BlockSpec((BLK, EMBED), lambda i: (i, 0)),
            pl.BlockSpec((EMBED, PACKED), lambda i: (0, 0)),
            pl.BlockSpec((EMBED, PACKED), lambda i: (0, 0)),
        ],
        out_specs=pl.BlockSpec((BLK, PACKED), lambda i: (i, 0)),
        out_shape=jax.ShapeDtypeStruct((VOCAB, PACKED), jnp.int32),
    )(table, we, wo)


def _make_sc_pool():
    info = plsc.get_sparse_core_info()
    nc, ns = info.num_cores, info.num_subcores
    nw = nc * ns
    rpb = B // nw
    rpw = 3 * rpb

    mesh = plsc.VectorSubcoreMesh(core_axis_name="c", subcore_axis_name="s")

    @functools.partial(
        pl.kernel,
        mesh=mesh,
        out_type=jax.ShapeDtypeStruct((ROWS, HIDDEN), jnp.float32),
        scratch_types=[
            pltpu.VMEM((rpw, NCHUNK, CHUNK), jnp.int32),
            *[pltpu.VMEM((CHUNK, PACKED), jnp.int32) for _ in range(NBUF)],
            pltpu.VMEM((rpw, HIDDEN), jnp.float32),
            pltpu.VMEM((HIDDEN,), jnp.float32),
            *[pltpu.SemaphoreType.DMA for _ in range(NBUF)],
        ],
        compiler_params=pltpu.CompilerParams(
            use_tc_tiling_on_sc=False, needs_layout_passes=False),
    )
    def sc_pool(a_hbm, p_hbm, n_hbm, proj_hbm, bias_hbm, out_hbm,
                idx_v, g0, g1, g2, g3, out_v, bias_v, s0, s1, s2, s3):
        wid = lax.axis_index("s") * nc + lax.axis_index("c")
        gbufs = (g0, g1, g2, g3)
        sems = (s0, s1, s2, s3)
        pltpu.sync_copy(bias_hbm, bias_v)
        for t, ids_hbm in enumerate((a_hbm, p_hbm, n_hbm)):
            pltpu.async_copy(ids_hbm.at[pl.ds(wid * rpb, rpb)],
                             idx_v.at[pl.ds(t * rpb, rpb)], sems[t])
        for t in range(3):
            pltpu.make_async_copy(
                a_hbm.at[pl.ds(0, rpb)],
                idx_v.at[pl.ds(t * rpb, rpb)], sems[t]).wait()
        for c in range(NBUF - 1):
            pltpu.async_copy(proj_hbm.at[idx_v.at[0, c]], gbufs[c], sems[c])
        dummy = proj_hbm.at[pl.ds(0, CHUNK)]

        def row_body(r, _):
            accs = tuple(jnp.zeros((16,), jnp.float32) for _ in range(4))
            for c in range(NCHUNK):
                gcur, scur = gbufs[c], sems[c]
                pltpu.make_async_copy(dummy, gcur, scur).wait()
                if c == 0:
                    pltpu.async_copy(
                        proj_hbm.at[idx_v.at[r, NCHUNK - 1]],
                        gbufs[NCHUNK - 1], sems[NCHUNK - 1])
                else:
                    @pl.when(r + 1 < rpw)
                    def _():
                        pltpu.async_copy(
                            proj_hbm.at[idx_v.at[r + 1, c - 1]],
                            gbufs[c - 1], sems[c - 1])

                @plsc.parallel_loop(0, CHUNK, unroll=16, carry=accs)
                def accs(i, carry):
                    a0, a1, a2, a3 = carry
                    w0 = gcur[i, pl.ds(0, 16)]
                    w1 = gcur[i, pl.ds(16, 16)]
                    a0 += plsc.bitcast(w0 << 16, jnp.float32)
                    a1 += plsc.bitcast(w0 & jnp.int32(-65536), jnp.float32)
                    a2 += plsc.bitcast(w1 << 16, jnp.float32)
                    a3 += plsc.bitcast(w1 & jnp.int32(-65536), jnp.float32)
                    return a0, a1, a2, a3

            inv = jnp.float32(1.0 / S)
            for q in range(4):
                val = jnp.maximum(
                    accs[q] * inv + bias_v[pl.ds(q * 16, 16)], 0.0)
                out_v[r, pl.ds(q * 16, 16)] = val
            return 0

        lax.fori_loop(0, rpw, row_body, 0)
        for t in range(3):
            pltpu.sync_copy(out_v.at[pl.ds(t * rpb, rpb)],
                            out_hbm.at[pl.ds(t * B + wid * rpb, rpb)])

    return sc_pool


def kernel(anchor_input_ids, anchor_attention_mask,
           positive_input_ids, positive_attention_mask,
           negative_input_ids, negative_attention_mask,
           emb_table, fc_W, fc_b):
    def prep(ids):
        return ids.astype(jnp.int32).reshape(B, NCHUNK, CHUNK)

    we = jnp.concatenate([fc_W[:, 0:16], fc_W[:, 32:48]], axis=1)
    wo = jnp.concatenate([fc_W[:, 16:32], fc_W[:, 48:64]], axis=1)
    proj = _project_table(emb_table, we, wo)
    return _make_sc_pool()(
        prep(anchor_input_ids), prep(positive_input_ids),
        prep(negative_input_ids), proj, fc_b)


# speedup vs baseline: 37.3691x; 1.2091x over previous
import functools

import jax
import jax.numpy as jnp
from jax import lax
from jax.experimental import pallas as pl
from jax.experimental.pallas import tpu as pltpu
from jax.experimental.pallas import tpu_sc as plsc

VOCAB = 30522
EMBED = 128
HIDDEN = 64
PACKED = HIDDEN // 2
B = 1024
S = 512
ROWS = 3 * B
CHUNK = 128
NCHUNK = S // CHUNK
NBUF = 8

BLK = 2048


def _round_to_bf16_bits(x):
    u = lax.bitcast_convert_type(x, jnp.int32)
    r = u + jnp.int32(0x7FFF) + ((u >> 16) & jnp.int32(1))
    return r & jnp.int32(-65536)


def _proj_body(tab_ref, we_ref, wo_ref, out_ref):
    pe = jnp.dot(tab_ref[...], we_ref[...],
                 preferred_element_type=jnp.float32)
    po = jnp.dot(tab_ref[...], wo_ref[...],
                 preferred_element_type=jnp.float32)
    lo = lax.shift_right_logical(_round_to_bf16_bits(pe), 16)
    hi = _round_to_bf16_bits(po)
    out_ref[...] = hi | lo


def _project_table(table, we, wo):
    grid = (VOCAB + BLK - 1) // BLK
    return pl.pallas_call(
        _proj_body,
        grid=(grid,),
        in_specs=[
            pl.BlockSpec((BLK, EMBED), lambda i: (i, 0)),
            pl.BlockSpec((EMBED, PACKED), lambda i: (0, 0)),
            pl.BlockSpec((EMBED, PACKED), lambda i: (0, 0)),
        ],
        out_specs=pl.BlockSpec((BLK, PACKED), lambda i: (i, 0)),
        out_shape=jax.ShapeDtypeStruct((VOCAB, PACKED), jnp.int32),
    )(table, we, wo)


def _make_sc_pool():
    info = plsc.get_sparse_core_info()
    nc, ns = info.num_cores, info.num_subcores
    nw = nc * ns
    rpb = B // nw
    rpw = 3 * rpb

    mesh = plsc.VectorSubcoreMesh(core_axis_name="c", subcore_axis_name="s")

    @functools.partial(
        pl.kernel,
        mesh=mesh,
        out_type=jax.ShapeDtypeStruct((ROWS, HIDDEN), jnp.float32),
        scratch_types=[
            pltpu.VMEM((rpw, NCHUNK, CHUNK), jnp.int32),
            *[pltpu.VMEM((CHUNK, PACKED), jnp.int32) for _ in range(NBUF)],
            pltpu.VMEM((rpw, HIDDEN), jnp.float32),
            pltpu.VMEM((HIDDEN,), jnp.float32),
            *[pltpu.SemaphoreType.DMA for _ in range(NBUF)],
        ],
        compiler_params=pltpu.CompilerParams(
            use_tc_tiling_on_sc=False, needs_layout_passes=False),
    )
    def sc_pool(a_hbm, p_hbm, n_hbm, proj_hbm, bias_hbm, out_hbm,
                idx_v, g0, g1, g2, g3, g4, g5, g6, g7, out_v, bias_v,
                s0, s1, s2, s3, s4, s5, s6, s7):
        wid = lax.axis_index("s") * nc + lax.axis_index("c")
        gbufs = (g0, g1, g2, g3, g4, g5, g6, g7)
        sems = (s0, s1, s2, s3, s4, s5, s6, s7)
        pltpu.sync_copy(bias_hbm, bias_v)
        for t, ids_hbm in enumerate((a_hbm, p_hbm, n_hbm)):
            pltpu.async_copy(ids_hbm.at[pl.ds(wid * rpb, rpb)],
                             idx_v.at[pl.ds(t * rpb, rpb)], sems[t])
        for t in range(3):
            pltpu.make_async_copy(
                a_hbm.at[pl.ds(0, rpb)],
                idx_v.at[pl.ds(t * rpb, rpb)], sems[t]).wait()
        for k in range(NBUF - 1):
            pltpu.async_copy(proj_hbm.at[idx_v.at[k // NCHUNK, k % NCHUNK]],
                             gbufs[k], sems[k])
        dummy = proj_hbm.at[pl.ds(0, CHUNK)]

        def group_body(g, _):
            base_r = g * 2
            for half in range(2):
                r = base_r + half
                accs = tuple(jnp.zeros((16,), jnp.float32) for _ in range(4))
                for c in range(NCHUNK):
                    m = half * NCHUNK + c
                    gcur, scur = gbufs[m], sems[m]
                    pltpu.make_async_copy(dummy, gcur, scur).wait()
                    tr_off = (m + NBUF - 1) // NCHUNK
                    tc = (m + NBUF - 1) % NCHUNK
                    tb = (m + NBUF - 1) % NBUF
                    tr = base_r + tr_off

                    @pl.when(tr < rpw)
                    def _():
                        pltpu.async_copy(
                            proj_hbm.at[idx_v.at[tr, tc]],
                            gbufs[tb], sems[tb])

                    @plsc.parallel_loop(0, CHUNK, unroll=16, carry=accs)
                    def accs(i, carry):
                        a0, a1, a2, a3 = carry
                        w0 = gcur[i, pl.ds(0, 16)]
                        w1 = gcur[i, pl.ds(16, 16)]
                        a0 += plsc.bitcast(w0 << 16, jnp.float32)
                        a1 += plsc.bitcast(w0 & jnp.int32(-65536), jnp.float32)
                        a2 += plsc.bitcast(w1 << 16, jnp.float32)
                        a3 += plsc.bitcast(w1 & jnp.int32(-65536), jnp.float32)
                        return a0, a1, a2, a3

                inv = jnp.float32(1.0 / S)
                for q in range(4):
                    val = jnp.maximum(
                        accs[q] * inv + bias_v[pl.ds(q * 16, 16)], 0.0)
                    out_v[r, pl.ds(q * 16, 16)] = val
            return 0

        lax.fori_loop(0, rpw // 2, group_body, 0)
        for t in range(3):
            pltpu.sync_copy(out_v.at[pl.ds(t * rpb, rpb)],
                            out_hbm.at[pl.ds(t * B + wid * rpb, rpb)])

    return sc_pool


def kernel(anchor_input_ids, anchor_attention_mask,
           positive_input_ids, positive_attention_mask,
           negative_input_ids, negative_attention_mask,
           emb_table, fc_W, fc_b):
    def prep(ids):
        return ids.astype(jnp.int32).reshape(B, NCHUNK, CHUNK)

    we = jnp.concatenate([fc_W[:, 0:16], fc_W[:, 32:48]], axis=1)
    wo = jnp.concatenate([fc_W[:, 16:32], fc_W[:, 48:64]], axis=1)
    proj = _project_table(emb_table, we, wo)
    return _make_sc_pool()(
        prep(anchor_input_ids), prep(positive_input_ids),
        prep(negative_input_ids), proj, fc_b)
